# baseline (speedup 1.0000x reference)
import functools

import jax
import jax.numpy as jnp
from jax import lax
from jax.experimental import pallas as pl
from jax.experimental.pallas import tpu as pltpu
from jax.experimental.pallas import tpu_sc as plsc

B, T, C, D = 2, 1, 2048, 768
R, K, H = 16, 8, 12
DH = D // H
FF = 2048
D_IN, S_DIM = 384, 192
N = R * K
NTOT = B * N

NC, NS = 2, 16
NW = NC * NS
ROWS_PER_W = NTOT // NW
NEG = -1e30

CCH = 512
NCHUNK = C // CCH
NBUF = 4


def _routing_body(node_hbm, tem_ref, rc_ref, rs_ref, fw_ref, fb_ref,
                  fused_ref, fif_ref, tvf_ref, lossc_ref,
                  buf0, buf1, buf2, buf3, sems):
    bufs = (buf0, buf1, buf2, buf3)
    ntot_chunks = B * NCHUNK
    cps = [None] * ntot_chunks

    def start(i):
        if i < ntot_chunks and cps[i] is None:
            b, cc = divmod(i, NCHUNK)
            cp = pltpu.make_async_copy(
                node_hbm.at[b, 0, pl.ds(cc * CCH, CCH)],
                bufs[i % NBUF], sems.at[i % NBUF])
            cp.start()
            cps[i] = cp

    start(0)
    start(1)
    start(2)
    fb = fb_ref[...]
    rc = rc_ref[...]
    rs = rs_ref[...]
    loss_acc = 0.0
    for b in range(B):
        tem = tem_ref[b:b + 1, :]
        fused_b = (
            jnp.dot(rc, fw_ref[0:D_IN, :], preferred_element_type=jnp.float32)
            + jnp.dot(tem, fw_ref[D_IN:D - S_DIM, :],
                      preferred_element_type=jnp.float32)
            + jnp.dot(rs, fw_ref[D - S_DIM:D, :],
                      preferred_element_type=jnp.float32)
            + fb
        )
        fused_ref[b] = fused_b

        lparts = []
        for cc in range(NCHUNK):
            i = b * NCHUNK + cc
            start(i + 3)
            cps[i].wait()
            lparts.append(lax.dot_general(
                fused_b, bufs[i % NBUF][...], (((1,), (1,)), ((), ())),
                preferred_element_type=jnp.float32))
        logits = jnp.concatenate(lparts, axis=1)
        m = jnp.max(logits, axis=1, keepdims=True)
        e = jnp.exp(logits - m)
        zden = jnp.sum(e, axis=1, keepdims=True)
        iota_c = lax.broadcasted_iota(jnp.int32, (R, C), 1)
        cur = e
        for k in range(K):
            v = jnp.max(cur, axis=1, keepdims=True)
            is_max = cur == v
            idx = jnp.min(jnp.where(is_max, iota_c, C), axis=1,
                          keepdims=True)
            fif_ref[pl.ds(b * N + k * R, R)] = jnp.reshape(idx + b * C, (R,))
            tvf_ref[pl.ds(b * N + k * R, R)] = jnp.reshape(v / zden, (R,))
            cur = jnp.where(iota_c == idx, -1.0, cur)

        nrm = jnp.sqrt(jnp.sum(fused_b * fused_b, axis=1, keepdims=True))
        z = fused_b / jnp.maximum(nrm, 1e-12)
        s16 = lax.dot_general(z, z, (((1,), (1,)), ((), ())),
                              preferred_element_type=jnp.float32) / 0.3
        e16 = jnp.exp(s16)
        eye = (lax.broadcasted_iota(jnp.int32, (R, R), 0)
               == lax.broadcasted_iota(jnp.int32, (R, R), 1))
        neg = jnp.sum(jnp.where(eye, 0.0, e16), axis=1, keepdims=True)
        de = jnp.exp(jnp.float32(1.0 / 0.3))
        loss_b = -jnp.log(de / (de + neg) + 1e-08)
        loss_acc = loss_acc + jnp.sum(loss_b)
    lossc_ref[...] = jnp.broadcast_to(loss_acc / (B * R), (1, 1))


def _routing_call(node_4d, tem, rc, rs, fw, fb, interpret=False):
    return pl.pallas_call(
        _routing_body,
        out_shape=(
            jax.ShapeDtypeStruct((B, R, D), jnp.float32),
            jax.ShapeDtypeStruct((NTOT,), jnp.int32),
            jax.ShapeDtypeStruct((NTOT,), jnp.float32),
            jax.ShapeDtypeStruct((1, 1), jnp.float32),
        ),
        in_specs=[
            pl.BlockSpec(memory_space=pl.ANY),
            pl.BlockSpec(memory_space=pltpu.VMEM),
            pl.BlockSpec(memory_space=pltpu.VMEM),
            pl.BlockSpec(memory_space=pltpu.VMEM),
            pl.BlockSpec(memory_space=pltpu.VMEM),
            pl.BlockSpec(memory_space=pltpu.VMEM),
        ],
        scratch_shapes=[
            pltpu.VMEM((CCH, D), jnp.float32),
            pltpu.VMEM((CCH, D), jnp.float32),
            pltpu.VMEM((CCH, D), jnp.float32),
            pltpu.VMEM((CCH, D), jnp.float32),
            pltpu.SemaphoreType.DMA((NBUF,)),
        ],
        interpret=interpret,
    )(node_4d, tem, rc, rs, fw, fb)


def _sc_gather(table, flat_idx):
    mesh = plsc.VectorSubcoreMesh(core_axis_name="c", subcore_axis_name="s",
                                  num_cores=NC, num_subcores=NS)

    @functools.partial(
        pl.kernel, mesh=mesh,
        out_type=jax.ShapeDtypeStruct((NTOT, D), jnp.float32),
        scratch_types=[
            pltpu.VMEM((ROWS_PER_W,), jnp.int32),
            pltpu.VMEM((ROWS_PER_W, D), jnp.float32),
            pltpu.SemaphoreType.DMA,
        ],
    )
    def gather_kernel(table_hbm, idx_hbm, out_hbm, idx_v, rows_v, sem):
        wid = lax.axis_index("s") * NC + lax.axis_index("c")
        base = wid * ROWS_PER_W
        pltpu.sync_copy(idx_hbm.at[pl.ds(base, ROWS_PER_W)], idx_v)
        pltpu.async_copy(table_hbm.at[idx_v], rows_v, sem).wait()
        pltpu.sync_copy(rows_v, out_hbm.at[pl.ds(base, ROWS_PER_W)])

    return gather_kernel(table, flat_idx)


def _softmax_rows(x):
    m = jnp.max(x, axis=1, keepdims=True)
    e = jnp.exp(x - m)
    return e / jnp.sum(e, axis=1, keepdims=True)


def _logsumexp_rows(x):
    m = jnp.max(x, axis=1, keepdims=True)
    return m + jnp.log(jnp.sum(jnp.exp(x - m), axis=1, keepdims=True))


def _ln_rows(x, g, b):
    mu = jnp.mean(x, axis=1, keepdims=True)
    xc = x - mu
    v = jnp.mean(xc * xc, axis=1, keepdims=True)
    return xc / jnp.sqrt(v + 1e-5) * g + b


def _main_body(g_ref, fused_ref, fif_ref, tvf_ref, lossc_ref,
               qw_h, qb_ref, kw_h, kb_ref, vw_h, vb_ref,
               ow_h, ob_ref, w1_h, b1_ref, w2_h, b2_ref,
               g1_ref, be1_ref, g2_ref, be2_ref,
               updw_ref, loss_ref,
               qw_v, kw_v, vw_v, ow_v, w1_v, w2_v, sems):
    bf = jnp.bfloat16
    cps = []
    for i, (h_ref, v_ref) in enumerate([(qw_h, qw_v), (kw_h, kw_v),
                                        (vw_h, vw_v), (ow_h, ow_v),
                                        (w1_h, w1_v), (w2_h, w2_v)]):
        cp = pltpu.make_async_copy(h_ref, v_ref, sems.at[i])
        cp.start()
        cps.append(cp)

    X = g_ref[...]
    Xb = X.astype(bf)

    i0 = lax.broadcasted_iota(jnp.int32, (NTOT, NTOT), 0)
    i1 = lax.broadcasted_iota(jnp.int32, (NTOT, NTOT), 1)
    same_grp = ((i0 % R) == (i1 % R)) & ((i0 // N) == (i1 // N))
    bias = jnp.where(same_grp, 0.0, NEG)
    scale = 1.0 / (DH ** 0.5)

    fi = fif_ref[...]
    tv = tvf_ref[...]
    fi_col = jnp.reshape(fi, (NTOT, 1))
    fi_row = jnp.reshape(fi, (1, NTOT))
    tv_col = jnp.reshape(tv, (NTOT, 1))
    tv_row = jnp.reshape(tv, (1, NTOT))
    same = fi_col == fi_row
    total = jnp.sum(jnp.where(same, tv_row, 0.0), axis=1, keepdims=True)
    w_col = tv_col / total

    acc = 0.0
    pos_mask = (lax.broadcasted_iota(jnp.int32, (R, N), 1) % R
                == lax.broadcasted_iota(jnp.int32, (R, N), 0))
    pos_bias = jnp.where(pos_mask, 0.0, NEG)
    for b in range(B):
        f_b = fused_ref[b]
        fn = f_b / jnp.maximum(
            jnp.sqrt(jnp.sum(f_b * f_b, axis=1, keepdims=True)), 1e-12)
        g_b = X[b * N:(b + 1) * N, :]
        gn = g_b / jnp.maximum(
            jnp.sqrt(jnp.sum(g_b * g_b, axis=1, keepdims=True)), 1e-12)
        logits = lax.dot_general(fn, gn, (((1,), (1,)), ((), ())),
                                 preferred_element_type=jnp.float32) / 0.1
        lpp = _logsumexp_rows(logits + pos_bias)
        lpa = _logsumexp_rows(logits)
        acc = acc + jnp.sum(lpa - lpp)
    loss_ref[...] = jnp.broadcast_to(acc / (B * R), (1, 1)) + lossc_ref[...]

    cps[0].wait()
    q = jnp.dot(Xb, qw_v[...].astype(bf),
                preferred_element_type=jnp.float32) + qb_ref[...]
    cps[1].wait()
    k = jnp.dot(Xb, kw_v[...].astype(bf),
                preferred_element_type=jnp.float32) + kb_ref[...]
    cps[2].wait()
    v = jnp.dot(Xb, vw_v[...].astype(bf),
                preferred_element_type=jnp.float32) + vb_ref[...]

    outs = []
    for h in range(H):
        sl = slice(h * DH, (h + 1) * DH)
        s = lax.dot_general(q[:, sl].astype(bf), k[:, sl].astype(bf),
                            (((1,), (1,)), ((), ())),
                            preferred_element_type=jnp.float32)
        p = _softmax_rows(s * scale + bias)
        outs.append(jnp.dot(p.astype(bf), v[:, sl].astype(bf),
                            preferred_element_type=jnp.float32))
    attn_out = jnp.concatenate(outs, axis=1)
    cps[3].wait()
    attn_out = jnp.dot(attn_out.astype(bf), ow_v[...].astype(bf),
                       preferred_element_type=jnp.float32) + ob_ref[...]

    y = _ln_rows(X + attn_out, g1_ref[...], be1_ref[...])
    cps[4].wait()
    h1 = jnp.maximum(jnp.dot(y.astype(bf), w1_v[...].astype(bf),
                             preferred_element_type=jnp.float32) + b1_ref[...], 0.0)
    cps[5].wait()
    y2 = jnp.dot(h1.astype(bf), w2_v[...].astype(bf),
                 preferred_element_type=jnp.float32) + b2_ref[...]
    upd = _ln_rows(y + y2, g2_ref[...], be2_ref[...])

    upd_w = upd * w_col
    updw_ref[...] = jnp.dot(same.astype(jnp.float32), upd_w,
                            preferred_element_type=jnp.float32)


def _main_call(gathered, fused, fi_flat, tv_flat, loss_c, wts,
               interpret=False):
    big = {0, 2, 4, 6, 8, 10}
    in_specs = (
        [pl.BlockSpec(memory_space=pltpu.VMEM)] * 5
        + [pl.BlockSpec(memory_space=pl.ANY) if i in big
           else pl.BlockSpec(memory_space=pltpu.VMEM) for i in range(16)]
    )
    return pl.pallas_call(
        _main_body,
        out_shape=(
            jax.ShapeDtypeStruct((NTOT, D), jnp.float32),
            jax.ShapeDtypeStruct((1, 1), jnp.float32),
        ),
        in_specs=in_specs,
        scratch_shapes=[
            pltpu.VMEM((D, D), jnp.float32),
            pltpu.VMEM((D, D), jnp.float32),
            pltpu.VMEM((D, D), jnp.float32),
            pltpu.VMEM((D, D), jnp.float32),
            pltpu.VMEM((D, FF), jnp.float32),
            pltpu.VMEM((FF, D), jnp.float32),
            pltpu.SemaphoreType.DMA((6,)),
        ],
        interpret=interpret,
    )(gathered, fused, fi_flat, tv_flat, loss_c, *wts)


def _sc_scatter(rows, flat_idx):
    mesh = plsc.VectorSubcoreMesh(core_axis_name="c", subcore_axis_name="s",
                                  num_cores=NC, num_subcores=NS)
    rows_per_tile = N // NS
    c_per_tile = C // NS
    ZROWS = 16

    @functools.partial(
        pl.kernel, mesh=mesh,
        out_type=jax.ShapeDtypeStruct((B * C, D), jnp.float32),
        scratch_types=[
            pltpu.VMEM((rows_per_tile,), jnp.int32),
            pltpu.VMEM((rows_per_tile, D), jnp.float32),
            pltpu.VMEM((ZROWS, D), jnp.float32),
            pltpu.SemaphoreType.DMA,
            pltpu.SemaphoreType.DMA,
        ],
    )
    def scatter_kernel(rows_hbm, idx_hbm, zeros_hbm, out_hbm, idx_v, rows_v,
                       zbuf, sem_z, sem_g):
        cid = lax.axis_index("c")
        sid = lax.axis_index("s")
        base = cid * N + sid * rows_per_tile
        cp_i = pltpu.async_copy(idx_hbm.at[pl.ds(base, rows_per_tile)],
                                idx_v, sem_g)
        cp_r = pltpu.async_copy(rows_hbm.at[pl.ds(base, rows_per_tile)],
                                rows_v, sem_g)
        pltpu.sync_copy(zeros_hbm, zbuf)
        row0 = cid * C + sid * c_per_tile
        zcps = [
            pltpu.async_copy(zbuf, out_hbm.at[pl.ds(row0 + t * ZROWS, ZROWS)],
                             sem_z)
            for t in range(c_per_tile // ZROWS)
        ]
        cp_i.wait()
        cp_r.wait()
        for cp in zcps:
            cp.wait()
        plsc.subcore_barrier()
        pltpu.async_copy(rows_v, out_hbm.at[idx_v], sem_g).wait()

    return scatter_kernel(rows, flat_idx,
                          jnp.zeros((ZROWS, D), jnp.float32))


def kernel(node_routing, tem_routing, routing_center, routing_spa,
           fuse_W, fuse_b, q_W, q_b, k_W, k_b, v_W, v_b, o_W, o_b,
           ff_W1, ff_b1, ff_W2, ff_b2, ln1_g, ln1_b, ln2_g, ln2_b):
    fused, fi_flat, tv_flat, loss_c = _routing_call(
        node_routing, tem_routing, routing_center, routing_spa,
        fuse_W, fuse_b)

    gathered = _sc_gather(node_routing.reshape(B * C, D), fi_flat)

    wts = (q_W, q_b, k_W, k_b, v_W, v_b, o_W, o_b,
           ff_W1, ff_b1, ff_W2, ff_b2, ln1_g, ln1_b, ln2_g, ln2_b)
    upd_w, loss = _main_call(gathered, fused, fi_flat, tv_flat, loss_c, wts)

    agg = _sc_scatter(upd_w, fi_flat)

    return agg.reshape(B, T, C, D), loss[0, 0]


# speedup vs baseline: 1.0798x; 1.0798x over previous
import functools

import jax
import jax.numpy as jnp
from jax import lax
from jax.experimental import pallas as pl
from jax.experimental.pallas import tpu as pltpu
from jax.experimental.pallas import tpu_sc as plsc

B, T, C, D = 2, 1, 2048, 768
R, K, H = 16, 8, 12
DH = D // H
FF = 2048
D_IN, S_DIM = 384, 192
N = R * K
NTOT = B * N

NC, NS = 2, 16
NW = NC * NS
ROWS_PER_W = NTOT // NW
NEG = -1e30

CCH = 512
NCHUNK = C // CCH
NBUF = 4


def _routing_body(node_hbm, tem_ref, rc_ref, rs_ref, fw_ref, fb_ref,
                  fused_ref, fif_ref, tvf_ref, lossc_ref,
                  buf0, buf1, buf2, buf3, sems):
    bufs = (buf0, buf1, buf2, buf3)
    ntot_chunks = B * NCHUNK
    cps = [None] * ntot_chunks

    def start(i):
        if i < ntot_chunks and cps[i] is None:
            b, cc = divmod(i, NCHUNK)
            cp = pltpu.make_async_copy(
                node_hbm.at[b, 0, pl.ds(cc * CCH, CCH)],
                bufs[i % NBUF], sems.at[i % NBUF])
            cp.start()
            cps[i] = cp

    start(0)
    start(1)
    start(2)
    fb = fb_ref[...]
    rc = rc_ref[...]
    rs = rs_ref[...]
    loss_acc = 0.0
    for b in range(B):
        tem = tem_ref[b:b + 1, :]
        fused_b = (
            jnp.dot(rc, fw_ref[0:D_IN, :], preferred_element_type=jnp.float32)
            + jnp.dot(tem, fw_ref[D_IN:D - S_DIM, :],
                      preferred_element_type=jnp.float32)
            + jnp.dot(rs, fw_ref[D - S_DIM:D, :],
                      preferred_element_type=jnp.float32)
            + fb
        )
        fused_ref[b] = fused_b

        lparts = []
        for cc in range(NCHUNK):
            i = b * NCHUNK + cc
            start(i + 3)
            cps[i].wait()
            lparts.append(lax.dot_general(
                fused_b, bufs[i % NBUF][...], (((1,), (1,)), ((), ())),
                preferred_element_type=jnp.float32))
        logits = jnp.concatenate(lparts, axis=1)
        m = jnp.max(logits, axis=1, keepdims=True)
        e = jnp.exp(logits - m)
        zden = jnp.sum(e, axis=1, keepdims=True)
        iota_c = lax.broadcasted_iota(jnp.int32, (R, C), 1)
        cur = e
        for k in range(K):
            v = jnp.max(cur, axis=1, keepdims=True)
            is_max = cur == v
            idx = jnp.min(jnp.where(is_max, iota_c, C), axis=1,
                          keepdims=True)
            fif_ref[pl.ds(b * N + k * R, R)] = jnp.reshape(idx + b * C, (R,))
            tvf_ref[pl.ds(b * N + k * R, R)] = jnp.reshape(v / zden, (R,))
            cur = jnp.where(iota_c == idx, -1.0, cur)

        nrm = jnp.sqrt(jnp.sum(fused_b * fused_b, axis=1, keepdims=True))
        z = fused_b / jnp.maximum(nrm, 1e-12)
        s16 = lax.dot_general(z, z, (((1,), (1,)), ((), ())),
                              preferred_element_type=jnp.float32) / 0.3
        e16 = jnp.exp(s16)
        eye = (lax.broadcasted_iota(jnp.int32, (R, R), 0)
               == lax.broadcasted_iota(jnp.int32, (R, R), 1))
        neg = jnp.sum(jnp.where(eye, 0.0, e16), axis=1, keepdims=True)
        de = jnp.exp(jnp.float32(1.0 / 0.3))
        loss_b = -jnp.log(de / (de + neg) + 1e-08)
        loss_acc = loss_acc + jnp.sum(loss_b)
    lossc_ref[...] = jnp.broadcast_to(loss_acc / (B * R), (1, 1))


def _routing_call(node_4d, tem, rc, rs, fw, fb, interpret=False):
    return pl.pallas_call(
        _routing_body,
        out_shape=(
            jax.ShapeDtypeStruct((B, R, D), jnp.float32),
            jax.ShapeDtypeStruct((NTOT,), jnp.int32),
            jax.ShapeDtypeStruct((NTOT,), jnp.float32),
            jax.ShapeDtypeStruct((1, 1), jnp.float32),
        ),
        in_specs=[
            pl.BlockSpec(memory_space=pl.ANY),
            pl.BlockSpec(memory_space=pltpu.VMEM),
            pl.BlockSpec(memory_space=pltpu.VMEM),
            pl.BlockSpec(memory_space=pltpu.VMEM),
            pl.BlockSpec(memory_space=pltpu.VMEM),
            pl.BlockSpec(memory_space=pltpu.VMEM),
        ],
        scratch_shapes=[
            pltpu.VMEM((CCH, D), jnp.float32),
            pltpu.VMEM((CCH, D), jnp.float32),
            pltpu.VMEM((CCH, D), jnp.float32),
            pltpu.VMEM((CCH, D), jnp.float32),
            pltpu.SemaphoreType.DMA((NBUF,)),
        ],
        interpret=interpret,
    )(node_4d, tem, rc, rs, fw, fb)


def _sc_gather(table, flat_idx):
    mesh = plsc.VectorSubcoreMesh(core_axis_name="c", subcore_axis_name="s",
                                  num_cores=NC, num_subcores=NS)

    @functools.partial(
        pl.kernel, mesh=mesh,
        out_type=jax.ShapeDtypeStruct((NTOT, D), jnp.float32),
        scratch_types=[
            pltpu.VMEM((ROWS_PER_W,), jnp.int32),
            pltpu.VMEM((ROWS_PER_W, D), jnp.float32),
            pltpu.SemaphoreType.DMA,
        ],
    )
    def gather_kernel(table_hbm, idx_hbm, out_hbm, idx_v, rows_v, sem):
        wid = lax.axis_index("s") * NC + lax.axis_index("c")
        base = wid * ROWS_PER_W
        pltpu.sync_copy(idx_hbm.at[pl.ds(base, ROWS_PER_W)], idx_v)
        pltpu.async_copy(table_hbm.at[idx_v], rows_v, sem).wait()
        pltpu.sync_copy(rows_v, out_hbm.at[pl.ds(base, ROWS_PER_W)])

    return gather_kernel(table, flat_idx)


def _softmax_rows(x):
    m = jnp.max(x, axis=1, keepdims=True)
    e = jnp.exp(x - m)
    return e / jnp.sum(e, axis=1, keepdims=True)


def _logsumexp_rows(x):
    m = jnp.max(x, axis=1, keepdims=True)
    return m + jnp.log(jnp.sum(jnp.exp(x - m), axis=1, keepdims=True))


def _ln_rows(x, g, b):
    mu = jnp.mean(x, axis=1, keepdims=True)
    xc = x - mu
    v = jnp.mean(xc * xc, axis=1, keepdims=True)
    return xc / jnp.sqrt(v + 1e-5) * g + b


def _main_body(g_ref, fused_ref, fif_ref, tvf_ref, lossc_ref,
               qw_h, qb_ref, kw_h, kb_ref, vw_h, vb_ref,
               ow_h, ob_ref, w1_h, b1_ref, w2_h, b2_ref,
               g1_ref, be1_ref, g2_ref, be2_ref,
               updw_ref, loss_ref,
               qw_v, kw_v, vw_v, ow_v, w1_v, w2_v, sems):
    bf = jnp.bfloat16
    cps = []
    for i, (h_ref, v_ref) in enumerate([(qw_h, qw_v), (kw_h, kw_v),
                                        (vw_h, vw_v), (ow_h, ow_v),
                                        (w1_h, w1_v), (w2_h, w2_v)]):
        cp = pltpu.make_async_copy(h_ref, v_ref, sems.at[i])
        cp.start()
        cps.append(cp)

    X = g_ref[...]
    Xb = X.astype(bf)

    i0 = lax.broadcasted_iota(jnp.int32, (NTOT, NTOT), 0)
    i1 = lax.broadcasted_iota(jnp.int32, (NTOT, NTOT), 1)
    same_grp = ((i0 % R) == (i1 % R)) & ((i0 // N) == (i1 // N))
    bias = jnp.where(same_grp, 0.0, NEG)
    scale = 1.0 / (DH ** 0.5)

    fi = fif_ref[...]
    tv = tvf_ref[...]
    fi_col = jnp.reshape(fi, (NTOT, 1))
    fi_row = jnp.reshape(fi, (1, NTOT))
    tv_col = jnp.reshape(tv, (NTOT, 1))
    tv_row = jnp.reshape(tv, (1, NTOT))
    same = fi_col == fi_row
    total = jnp.sum(jnp.where(same, tv_row, 0.0), axis=1, keepdims=True)
    w_col = tv_col / total

    acc = 0.0
    pos_mask = (lax.broadcasted_iota(jnp.int32, (R, N), 1) % R
                == lax.broadcasted_iota(jnp.int32, (R, N), 0))
    pos_bias = jnp.where(pos_mask, 0.0, NEG)
    for b in range(B):
        f_b = fused_ref[b]
        fn = f_b / jnp.maximum(
            jnp.sqrt(jnp.sum(f_b * f_b, axis=1, keepdims=True)), 1e-12)
        g_b = X[b * N:(b + 1) * N, :]
        gn = g_b / jnp.maximum(
            jnp.sqrt(jnp.sum(g_b * g_b, axis=1, keepdims=True)), 1e-12)
        logits = lax.dot_general(fn, gn, (((1,), (1,)), ((), ())),
                                 preferred_element_type=jnp.float32) / 0.1
        lpp = _logsumexp_rows(logits + pos_bias)
        lpa = _logsumexp_rows(logits)
        acc = acc + jnp.sum(lpa - lpp)
    loss_ref[...] = jnp.broadcast_to(acc / (B * R), (1, 1)) + lossc_ref[...]

    cps[0].wait()
    q = jnp.dot(Xb, qw_v[...].astype(bf),
                preferred_element_type=jnp.float32) + qb_ref[...]
    cps[1].wait()
    k = jnp.dot(Xb, kw_v[...].astype(bf),
                preferred_element_type=jnp.float32) + kb_ref[...]
    cps[2].wait()
    v = jnp.dot(Xb, vw_v[...].astype(bf),
                preferred_element_type=jnp.float32) + vb_ref[...]

    outs = []
    for h in range(H):
        sl = slice(h * DH, (h + 1) * DH)
        s = lax.dot_general(q[:, sl].astype(bf), k[:, sl].astype(bf),
                            (((1,), (1,)), ((), ())),
                            preferred_element_type=jnp.float32)
        p = _softmax_rows(s * scale + bias)
        outs.append(jnp.dot(p.astype(bf), v[:, sl].astype(bf),
                            preferred_element_type=jnp.float32))
    attn_out = jnp.concatenate(outs, axis=1)
    cps[3].wait()
    attn_out = jnp.dot(attn_out.astype(bf), ow_v[...].astype(bf),
                       preferred_element_type=jnp.float32) + ob_ref[...]

    y = _ln_rows(X + attn_out, g1_ref[...], be1_ref[...])
    cps[4].wait()
    h1 = jnp.maximum(jnp.dot(y.astype(bf), w1_v[...].astype(bf),
                             preferred_element_type=jnp.float32) + b1_ref[...], 0.0)
    cps[5].wait()
    y2 = jnp.dot(h1.astype(bf), w2_v[...].astype(bf),
                 preferred_element_type=jnp.float32) + b2_ref[...]
    upd = _ln_rows(y + y2, g2_ref[...], be2_ref[...])

    upd_w = upd * w_col
    updw_ref[...] = jnp.dot(same.astype(jnp.float32), upd_w,
                            preferred_element_type=jnp.float32)


def _main_call(gathered, fused, fi_flat, tv_flat, loss_c, wts,
               interpret=False):
    big = {0, 2, 4, 6, 8, 10}
    in_specs = (
        [pl.BlockSpec(memory_space=pltpu.VMEM)] * 5
        + [pl.BlockSpec(memory_space=pl.ANY) if i in big
           else pl.BlockSpec(memory_space=pltpu.VMEM) for i in range(16)]
    )
    return pl.pallas_call(
        _main_body,
        out_shape=(
            jax.ShapeDtypeStruct((NTOT, D), jnp.float32),
            jax.ShapeDtypeStruct((1, 1), jnp.float32),
        ),
        in_specs=in_specs,
        scratch_shapes=[
            pltpu.VMEM((D, D), jnp.float32),
            pltpu.VMEM((D, D), jnp.float32),
            pltpu.VMEM((D, D), jnp.float32),
            pltpu.VMEM((D, D), jnp.float32),
            pltpu.VMEM((D, FF), jnp.float32),
            pltpu.VMEM((FF, D), jnp.float32),
            pltpu.SemaphoreType.DMA((6,)),
        ],
        interpret=interpret,
    )(gathered, fused, fi_flat, tv_flat, loss_c, *wts)


def _sc_scatter(rows, flat_idx):
    mesh = plsc.VectorSubcoreMesh(core_axis_name="c", subcore_axis_name="s",
                                  num_cores=NC, num_subcores=NS)
    rows_per_tile = N // NS
    c_per_tile = C // NS
    ZROWS = 16

    @functools.partial(
        pl.kernel, mesh=mesh,
        out_type=jax.ShapeDtypeStruct((B * C, D), jnp.float32),
        scratch_types=[
            pltpu.VMEM((rows_per_tile,), jnp.int32),
            pltpu.VMEM((rows_per_tile, D), jnp.float32),
            pltpu.VMEM((ZROWS, D), jnp.float32),
            pltpu.SemaphoreType.DMA,
            pltpu.SemaphoreType.DMA,
        ],
    )
    def scatter_kernel(rows_hbm, idx_hbm, out_hbm, idx_v, rows_v,
                       zbuf, sem_z, sem_g):
        cid = lax.axis_index("c")
        sid = lax.axis_index("s")
        base = cid * N + sid * rows_per_tile
        cp_i = pltpu.async_copy(idx_hbm.at[pl.ds(base, rows_per_tile)],
                                idx_v, sem_g)
        cp_r = pltpu.async_copy(rows_hbm.at[pl.ds(base, rows_per_tile)],
                                rows_v, sem_g)
        zeros16 = jnp.zeros((16,), jnp.float32)
        for i in range(ZROWS):
            for j in range(D // 16):
                zbuf[i, pl.ds(j * 16, 16)] = zeros16
        row0 = cid * C + sid * c_per_tile
        zcps = [
            pltpu.async_copy(zbuf, out_hbm.at[pl.ds(row0 + t * ZROWS, ZROWS)],
                             sem_z)
            for t in range(c_per_tile // ZROWS)
        ]
        cp_i.wait()
        cp_r.wait()
        for cp in zcps:
            cp.wait()
        plsc.subcore_barrier()
        pltpu.async_copy(rows_v, out_hbm.at[idx_v], sem_g).wait()

    return scatter_kernel(rows, flat_idx)


def kernel(node_routing, tem_routing, routing_center, routing_spa,
           fuse_W, fuse_b, q_W, q_b, k_W, k_b, v_W, v_b, o_W, o_b,
           ff_W1, ff_b1, ff_W2, ff_b2, ln1_g, ln1_b, ln2_g, ln2_b):
    fused, fi_flat, tv_flat, loss_c = _routing_call(
        node_routing, tem_routing, routing_center, routing_spa,
        fuse_W, fuse_b)

    gathered = _sc_gather(node_routing.reshape(B * C, D), fi_flat)

    wts = (q_W, q_b, k_W, k_b, v_W, v_b, o_W, o_b,
           ff_W1, ff_b1, ff_W2, ff_b2, ln1_g, ln1_b, ln2_g, ln2_b)
    upd_w, loss = _main_call(gathered, fused, fi_flat, tv_flat, loss_c, wts)

    agg = _sc_scatter(upd_w, fi_flat)

    return agg.reshape(B, T, C, D), loss[0, 0]
